# baseline (speedup 1.0000x reference)
import functools

import jax
import jax.numpy as jnp
from jax import lax
from jax.experimental import pallas as pl
from jax.experimental.pallas import tpu as pltpu
from jax.experimental.pallas import tpu_sc as plsc

N = 10000
E = 160000
D = 256
DH = 128
B = 512
EC = 128
NECHUNK = E // EC
PC = 80
NPCHUNK = N // PC
NSUB = 16
RB = 1000

_mesh = plsc.VectorSubcoreMesh(core_axis_name="c", subcore_axis_name="s")
_f32 = jnp.float32


def _dot3(a, b):
    return jax.lax.dot_general(a.astype(jnp.bfloat16), b.astype(jnp.bfloat16),
                               (((1,), (0,)), ((), ())),
                               preferred_element_type=jnp.float32)


def _layer_body(a0_ref, a1_ref, hp_ref, wn_ref, wr_ref, b_ref,
                h_ref, h0_ref, h1_ref):
    agg = jnp.concatenate([a0_ref[...], a1_ref[...]], axis=1)
    h = jnp.maximum(_dot3(agg, wn_ref[...]) + b_ref[...]
                    + _dot3(hp_ref[...], wr_ref[...]), 0.0)
    h_ref[...] = h
    h0_ref[...] = h[:, :DH]
    h1_ref[...] = h[:, DH:]


def _layer3_body(a0_ref, a1_ref, hp_ref, wn_ref, wr_ref, b_ref, h_ref):
    agg = jnp.concatenate([a0_ref[...], a1_ref[...]], axis=1)
    h_ref[...] = jnp.maximum(_dot3(agg, wn_ref[...]) + b_ref[...]
                             + _dot3(hp_ref[...], wr_ref[...]), 0.0)


def _split_body(x_ref, x0_ref, x1_ref):
    x0_ref[...] = x_ref[:, :DH]
    x1_ref[...] = x_ref[:, DH:]


def _row_spec(w):
    return pl.BlockSpec((RB, w), lambda i: (i, 0))


def _full_spec(a, b):
    return pl.BlockSpec((a, b), lambda i: (0, 0))


_layer = pl.pallas_call(
    _layer_body,
    grid=(N // RB,),
    in_specs=[_row_spec(DH), _row_spec(DH), _row_spec(D),
              _full_spec(D, D), _full_spec(D, D), _full_spec(1, D)],
    out_specs=[_row_spec(D), _row_spec(DH), _row_spec(DH)],
    out_shape=[jax.ShapeDtypeStruct((N, D), _f32),
               jax.ShapeDtypeStruct((N, DH), _f32),
               jax.ShapeDtypeStruct((N, DH), _f32)],
)

_layer3 = pl.pallas_call(
    _layer3_body,
    grid=(N // RB,),
    in_specs=[_row_spec(DH), _row_spec(DH), _row_spec(D),
              _full_spec(D, D), _full_spec(D, D), _full_spec(1, D)],
    out_specs=_row_spec(D),
    out_shape=jax.ShapeDtypeStruct((N, D), _f32),
)

_split = pl.pallas_call(
    _split_body,
    grid=(N // RB,),
    in_specs=[_row_spec(D)],
    out_specs=[_row_spec(DH), _row_spec(DH)],
    out_shape=[jax.ShapeDtypeStruct((N, DH), _f32),
               jax.ShapeDtypeStruct((N, DH), _f32)],
)


def _head_body(p0s, p1s, cs, rs, p0t, p1t, ct, rt, wlin, blin, wc, bc, lab,
               logits_ref, loss_ref):
    def emb(p0, p1, cnt, root):
        cm = jnp.maximum(cnt[...], 1.0)
        z = jnp.concatenate([p0[...] / cm, p1[...] / cm, root[...]], axis=1)
        return _dot3(z, wlin[...]) + blin[...]

    u = emb(p0s, p1s, cs, rs)
    v = emb(p0t, p1t, ct, rt)
    feats = jnp.concatenate([u, v, jnp.abs(u - v)], axis=1)
    logits = _dot3(feats, wc[...]) + bc[...]
    logits_ref[...] = logits
    mx = jnp.max(logits, axis=1, keepdims=True)
    lse = mx + jnp.log(jnp.sum(jnp.exp(logits - mx), axis=1, keepdims=True))
    logp = logits - lse
    sel = jnp.where(lab[...] == 0, logp[:, :1], logp[:, 1:2])
    loss_ref[...] = jnp.full((1, 1), -1.0 / B, jnp.float32) * jnp.sum(sel)


_head = pl.pallas_call(
    _head_body,
    in_specs=[pl.BlockSpec(memory_space=pltpu.VMEM)] * 13,
    out_specs=[pl.BlockSpec(memory_space=pltpu.VMEM)] * 2,
    out_shape=[jax.ShapeDtypeStruct((B, 2), _f32),
               jax.ShapeDtypeStruct((1, 1), _f32)],
)


_ROWS_PER_SUB = N // NSUB
CPS = 80
NCH2 = CPS * NSUB
EPAD = NCH2 * EC
NACC = N + 8
NGRP = NCH2 // 4


@functools.partial(
    pl.kernel, mesh=_mesh,
    out_type=(jax.ShapeDtypeStruct((N, DH), _f32),
              jax.ShapeDtypeStruct((N, DH), _f32)),
    scratch_types=[
        pltpu.VMEM((8, EC), jnp.int32),
        pltpu.VMEM((EC, DH), _f32),
        pltpu.VMEM_SHARED((NACC, DH), _f32),
        pltpu.SemaphoreType.DMA,
    ])
def _scatter(m0_hbm, m1_hbm, idx_hbm, zero_hbm, a0_hbm, a1_hbm,
             idxb, rows, acc, gsem):
    c = lax.axis_index("c")
    s = lax.axis_index("s")

    @pl.loop(s, NPCHUNK, step=NSUB)
    def _(k):
        pltpu.sync_copy(zero_hbm.at[pl.ds(k * PC, PC)],
                        acc.at[pl.ds(k * PC, PC)])

    plsc.subcore_barrier()

    def edge_loop(m_hbm):
        @pl.loop(0, CPS // 4)
        def _(t):
            base = 8 * (s * (CPS // 4) + t)
            pltpu.sync_copy(idx_hbm.at[pl.ds(base, 8)], idxb)
            for q in range(4):
                pltpu.async_copy(m_hbm.at[idxb.at[q]], rows, gsem).wait()
                pltpu.sync_copy(rows, acc.at[idxb.at[4 + q]], add=True)

    @pl.when(c == 0)
    def _():
        edge_loop(m0_hbm)

    @pl.when(c == 1)
    def _():
        edge_loop(m1_hbm)

    plsc.subcore_barrier()

    @pl.when(c == 0)
    def _():
        @pl.loop(s, NPCHUNK, step=NSUB)
        def _(k):
            pltpu.sync_copy(acc.at[pl.ds(k * PC, PC)],
                            a0_hbm.at[pl.ds(k * PC, PC)])

    @pl.when(c == 1)
    def _():
        @pl.loop(s, NPCHUNK, step=NSUB)
        def _(k):
            pltpu.sync_copy(acc.at[pl.ds(k * PC, PC)],
                            a1_hbm.at[pl.ds(k * PC, PC)])


_BROWS_PER_SUB = B // NSUB


@functools.partial(
    pl.kernel, mesh=_mesh,
    out_type=(jax.ShapeDtypeStruct((B, DH), _f32),
              jax.ShapeDtypeStruct((B, DH), _f32),
              jax.ShapeDtypeStruct((B, DH), _f32),
              jax.ShapeDtypeStruct((B, D), _f32)),
    scratch_types=[
        pltpu.VMEM((PC,), jnp.int32),
        pltpu.VMEM((PC, DH), _f32),
        pltpu.VMEM((PC, DH), _f32),
        pltpu.VMEM((16,), jnp.int32),
        pltpu.VMEM((16, D), _f32),
        pltpu.VMEM_SHARED((B, DH), _f32),
        pltpu.VMEM_SHARED((B, DH), _f32),
        pltpu.SemaphoreType.DMA,
    ])
def _pool(h_hbm, batch_hbm, root_hbm, ones_hbm, zero_hbm,
          p0_hbm, p1_hbm, cnt_hbm, remb_hbm,
          idx_v, rows_v, ones_v, ridx_v, rrows_v, acc, cacc, sem):
    c = lax.axis_index("c")
    s = lax.axis_index("s")
    b0 = s * _BROWS_PER_SUB
    pltpu.sync_copy(zero_hbm.at[pl.ds(b0, _BROWS_PER_SUB)],
                    acc.at[pl.ds(b0, _BROWS_PER_SUB)])
    pltpu.sync_copy(zero_hbm.at[pl.ds(b0, _BROWS_PER_SUB)],
                    cacc.at[pl.ds(b0, _BROWS_PER_SUB)])
    pltpu.sync_copy(ones_hbm, ones_v)
    plsc.subcore_barrier()

    w = s * 2 + c
    pltpu.sync_copy(root_hbm.at[pl.ds(w * 16, 16)], ridx_v)
    pltpu.async_copy(h_hbm.at[ridx_v], rrows_v, sem).wait()
    pltpu.sync_copy(rrows_v, remb_hbm.at[pl.ds(w * 16, 16)])

    def pool_loop(col):
        @pl.loop(s, NPCHUNK, step=NSUB)
        def _(k):
            base = k * PC
            pltpu.sync_copy(batch_hbm.at[pl.ds(base, PC)], idx_v)
            pltpu.sync_copy(h_hbm.at[pl.ds(base, PC), pl.ds(col, DH)], rows_v)
            pltpu.sync_copy(rows_v, acc.at[idx_v], add=True)
            pltpu.sync_copy(ones_v, cacc.at[idx_v], add=True)

    @pl.when(c == 0)
    def _():
        pool_loop(0)

    @pl.when(c == 1)
    def _():
        pool_loop(DH)

    plsc.subcore_barrier()

    @pl.when(c == 0)
    def _():
        pltpu.sync_copy(acc.at[pl.ds(b0, _BROWS_PER_SUB)],
                        p0_hbm.at[pl.ds(b0, _BROWS_PER_SUB)])
        pltpu.sync_copy(cacc.at[pl.ds(b0, _BROWS_PER_SUB)],
                        cnt_hbm.at[pl.ds(b0, _BROWS_PER_SUB)])

    @pl.when(c == 1)
    def _():
        pltpu.sync_copy(acc.at[pl.ds(b0, _BROWS_PER_SUB)],
                        p1_hbm.at[pl.ds(b0, _BROWS_PER_SUB)])


def kernel(x_s, edge_index_s, batch_s, root_n_id_s, x_t, edge_index_t,
           batch_t, root_n_id_t, labels, Wr1, Wn1, b1, Wr2, Wn2, b2,
           Wr3, Wn3, b3, Wlin, blin, Wc, bc):
    zeros = jnp.zeros((N, DH), _f32)
    ones = jnp.ones((PC, DH), _f32)
    b1r = b1.reshape(1, D)
    b2r = b2.reshape(1, D)
    b3r = b3.reshape(1, D)

    def gnn(x, ei, batch, root):
        pad = EPAD - E
        src = jnp.concatenate([ei[0], jnp.zeros((pad,), jnp.int32)])
        dst = jnp.concatenate([ei[1], jnp.full((pad,), N, jnp.int32)])
        def tile_major(a):
            return a.reshape(CPS, NSUB, EC).transpose(1, 0, 2).reshape(
                NSUB, CPS // 4, 4, EC)

        inter = jnp.concatenate(
            [tile_major(src), tile_major(dst)], axis=2).reshape(NGRP * 8, EC)
        x0, x1 = _split(x)
        a0, a1 = _scatter(x0, x1, inter, zeros)
        h, h0, h1 = _layer(a0, a1, x, Wn1, Wr1, b1r)
        a0, a1 = _scatter(h0, h1, inter, zeros)
        h, h0, h1 = _layer(a0, a1, h, Wn2, Wr2, b2r)
        a0, a1 = _scatter(h0, h1, inter, zeros)
        h3 = _layer3(a0, a1, h, Wn3, Wr3, b3r)
        return _pool(h3, batch, root, ones, zeros)

    p0s, p1s, cs, rs = gnn(x_s, edge_index_s, batch_s, root_n_id_s)
    p0t, p1t, ct, rt = gnn(x_t, edge_index_t, batch_t, root_n_id_t)

    logits, loss11 = _head(p0s, p1s, cs, rs, p0t, p1t, ct, rt,
                           Wlin, blin.reshape(1, D), Wc, bc.reshape(1, 2),
                           labels.reshape(B, 1))
    return (loss11[0, 0], logits)


# speedup vs baseline: 1.2847x; 1.2847x over previous
import functools

import jax
import jax.numpy as jnp
from jax import lax
from jax.experimental import pallas as pl
from jax.experimental.pallas import tpu as pltpu
from jax.experimental.pallas import tpu_sc as plsc

N = 10000
E = 160000
D = 256
DH = 128
B = 512
EC = 128
NECHUNK = E // EC
PC = 80
NPCHUNK = N // PC
NSUB = 16
RB = 1000

_mesh = plsc.VectorSubcoreMesh(core_axis_name="c", subcore_axis_name="s")
_f32 = jnp.float32


def _dot3(a, b):
    return jax.lax.dot_general(a.astype(jnp.bfloat16), b.astype(jnp.bfloat16),
                               (((1,), (0,)), ((), ())),
                               preferred_element_type=jnp.float32)


def _layer_body(a0_ref, a1_ref, hp_ref, wn_ref, wr_ref, b_ref,
                h_ref, h0_ref, h1_ref):
    agg = jnp.concatenate([a0_ref[...], a1_ref[...]], axis=1)
    h = jnp.maximum(_dot3(agg, wn_ref[...]) + b_ref[...]
                    + _dot3(hp_ref[...], wr_ref[...]), 0.0)
    h_ref[...] = h
    h0_ref[...] = h[:, :DH]
    h1_ref[...] = h[:, DH:]


def _layer3_body(a0_ref, a1_ref, hp_ref, wn_ref, wr_ref, b_ref, h_ref):
    agg = jnp.concatenate([a0_ref[...], a1_ref[...]], axis=1)
    h_ref[...] = jnp.maximum(_dot3(agg, wn_ref[...]) + b_ref[...]
                             + _dot3(hp_ref[...], wr_ref[...]), 0.0)


def _split_body(x_ref, x0_ref, x1_ref):
    x0_ref[...] = x_ref[:, :DH]
    x1_ref[...] = x_ref[:, DH:]


def _row_spec(w):
    return pl.BlockSpec((RB, w), lambda i: (i, 0))


def _full_spec(a, b):
    return pl.BlockSpec((a, b), lambda i: (0, 0))


_layer = pl.pallas_call(
    _layer_body,
    grid=(N // RB,),
    in_specs=[_row_spec(DH), _row_spec(DH), _row_spec(D),
              _full_spec(D, D), _full_spec(D, D), _full_spec(1, D)],
    out_specs=[_row_spec(D), _row_spec(DH), _row_spec(DH)],
    out_shape=[jax.ShapeDtypeStruct((N, D), _f32),
               jax.ShapeDtypeStruct((N, DH), _f32),
               jax.ShapeDtypeStruct((N, DH), _f32)],
)

_layer3 = pl.pallas_call(
    _layer3_body,
    grid=(N // RB,),
    in_specs=[_row_spec(DH), _row_spec(DH), _row_spec(D),
              _full_spec(D, D), _full_spec(D, D), _full_spec(1, D)],
    out_specs=_row_spec(D),
    out_shape=jax.ShapeDtypeStruct((N, D), _f32),
)

_split = pl.pallas_call(
    _split_body,
    grid=(N // RB,),
    in_specs=[_row_spec(D)],
    out_specs=[_row_spec(DH), _row_spec(DH)],
    out_shape=[jax.ShapeDtypeStruct((N, DH), _f32),
               jax.ShapeDtypeStruct((N, DH), _f32)],
)


def _head_body(p0s, p1s, cs, rs, p0t, p1t, ct, rt, wlin, blin, wc, bc, lab,
               logits_ref, loss_ref):
    def emb(p0, p1, cnt, root):
        cm = jnp.maximum(cnt[...], 1.0)
        z = jnp.concatenate([p0[...] / cm, p1[...] / cm, root[...]], axis=1)
        return _dot3(z, wlin[...]) + blin[...]

    u = emb(p0s, p1s, cs, rs)
    v = emb(p0t, p1t, ct, rt)
    feats = jnp.concatenate([u, v, jnp.abs(u - v)], axis=1)
    logits = _dot3(feats, wc[...]) + bc[...]
    logits_ref[...] = logits
    mx = jnp.max(logits, axis=1, keepdims=True)
    lse = mx + jnp.log(jnp.sum(jnp.exp(logits - mx), axis=1, keepdims=True))
    logp = logits - lse
    sel = jnp.where(lab[...] == 0, logp[:, :1], logp[:, 1:2])
    loss_ref[...] = jnp.full((1, 1), -1.0 / B, jnp.float32) * jnp.sum(sel)


_head = pl.pallas_call(
    _head_body,
    in_specs=[pl.BlockSpec(memory_space=pltpu.VMEM)] * 13,
    out_specs=[pl.BlockSpec(memory_space=pltpu.VMEM)] * 2,
    out_shape=[jax.ShapeDtypeStruct((B, 2), _f32),
               jax.ShapeDtypeStruct((1, 1), _f32)],
)


_ROWS_PER_SUB = N // NSUB
CPS = 80
NCH2 = CPS * NSUB
EPAD = NCH2 * EC
NACC = N + 8
NGRP = NCH2 // 4


@functools.partial(
    pl.kernel, mesh=_mesh,
    out_type=(jax.ShapeDtypeStruct((N, DH), _f32),
              jax.ShapeDtypeStruct((N, DH), _f32)),
    scratch_types=[
        pltpu.VMEM((EC,), jnp.int32),
        pltpu.VMEM((EC,), jnp.int32),
        pltpu.VMEM((EC, DH), _f32),
        pltpu.VMEM_SHARED((NACC, DH), _f32),
        pltpu.SemaphoreType.DMA,
    ])
def _scatter(m0_hbm, m1_hbm, src_hbm, dst_hbm, zero_hbm, a0_hbm, a1_hbm,
             idx_s, idx_d, rows, acc, gsem):
    c = lax.axis_index("c")
    s = lax.axis_index("s")

    @pl.loop(s, NPCHUNK, step=NSUB)
    def _(k):
        pltpu.sync_copy(zero_hbm.at[pl.ds(k * PC, PC)],
                        acc.at[pl.ds(k * PC, PC)])

    plsc.subcore_barrier()

    def edge_loop(m_hbm):
        @pl.loop(s, NECHUNK, step=NSUB)
        def _(k):
            base = k * EC
            pltpu.sync_copy(src_hbm.at[pl.ds(base, EC)], idx_s)
            pltpu.sync_copy(dst_hbm.at[pl.ds(base, EC)], idx_d)
            pltpu.async_copy(m_hbm.at[idx_s], rows, gsem).wait()
            pltpu.sync_copy(rows, acc.at[idx_d], add=True)

    @pl.when(c == 0)
    def _():
        edge_loop(m0_hbm)

    @pl.when(c == 1)
    def _():
        edge_loop(m1_hbm)

    plsc.subcore_barrier()

    @pl.when(c == 0)
    def _():
        @pl.loop(s, NPCHUNK, step=NSUB)
        def _(k):
            pltpu.sync_copy(acc.at[pl.ds(k * PC, PC)],
                            a0_hbm.at[pl.ds(k * PC, PC)])

    @pl.when(c == 1)
    def _():
        @pl.loop(s, NPCHUNK, step=NSUB)
        def _(k):
            pltpu.sync_copy(acc.at[pl.ds(k * PC, PC)],
                            a1_hbm.at[pl.ds(k * PC, PC)])


_BROWS_PER_SUB = B // NSUB


@functools.partial(
    pl.kernel, mesh=_mesh,
    out_type=(jax.ShapeDtypeStruct((B, DH), _f32),
              jax.ShapeDtypeStruct((B, DH), _f32),
              jax.ShapeDtypeStruct((B, DH), _f32),
              jax.ShapeDtypeStruct((B, D), _f32)),
    scratch_types=[
        pltpu.VMEM((PC,), jnp.int32),
        pltpu.VMEM((PC, DH), _f32),
        pltpu.VMEM((PC, DH), _f32),
        pltpu.VMEM((16,), jnp.int32),
        pltpu.VMEM((16, D), _f32),
        pltpu.VMEM_SHARED((B, DH), _f32),
        pltpu.VMEM_SHARED((B, DH), _f32),
        pltpu.SemaphoreType.DMA,
    ])
def _pool(h_hbm, batch_hbm, root_hbm, ones_hbm, zero_hbm,
          p0_hbm, p1_hbm, cnt_hbm, remb_hbm,
          idx_v, rows_v, ones_v, ridx_v, rrows_v, acc, cacc, sem):
    c = lax.axis_index("c")
    s = lax.axis_index("s")
    b0 = s * _BROWS_PER_SUB
    pltpu.sync_copy(zero_hbm.at[pl.ds(b0, _BROWS_PER_SUB)],
                    acc.at[pl.ds(b0, _BROWS_PER_SUB)])
    pltpu.sync_copy(zero_hbm.at[pl.ds(b0, _BROWS_PER_SUB)],
                    cacc.at[pl.ds(b0, _BROWS_PER_SUB)])
    pltpu.sync_copy(ones_hbm, ones_v)
    plsc.subcore_barrier()

    w = s * 2 + c
    pltpu.sync_copy(root_hbm.at[pl.ds(w * 16, 16)], ridx_v)
    pltpu.async_copy(h_hbm.at[ridx_v], rrows_v, sem).wait()
    pltpu.sync_copy(rrows_v, remb_hbm.at[pl.ds(w * 16, 16)])

    def pool_loop(col):
        @pl.loop(s, NPCHUNK, step=NSUB)
        def _(k):
            base = k * PC
            pltpu.sync_copy(batch_hbm.at[pl.ds(base, PC)], idx_v)
            pltpu.sync_copy(h_hbm.at[pl.ds(base, PC), pl.ds(col, DH)], rows_v)
            pltpu.sync_copy(rows_v, acc.at[idx_v], add=True)
            pltpu.sync_copy(ones_v, cacc.at[idx_v], add=True)

    @pl.when(c == 0)
    def _():
        pool_loop(0)

    @pl.when(c == 1)
    def _():
        pool_loop(DH)

    plsc.subcore_barrier()

    @pl.when(c == 0)
    def _():
        pltpu.sync_copy(acc.at[pl.ds(b0, _BROWS_PER_SUB)],
                        p0_hbm.at[pl.ds(b0, _BROWS_PER_SUB)])
        pltpu.sync_copy(cacc.at[pl.ds(b0, _BROWS_PER_SUB)],
                        cnt_hbm.at[pl.ds(b0, _BROWS_PER_SUB)])

    @pl.when(c == 1)
    def _():
        pltpu.sync_copy(acc.at[pl.ds(b0, _BROWS_PER_SUB)],
                        p1_hbm.at[pl.ds(b0, _BROWS_PER_SUB)])


def kernel(x_s, edge_index_s, batch_s, root_n_id_s, x_t, edge_index_t,
           batch_t, root_n_id_t, labels, Wr1, Wn1, b1, Wr2, Wn2, b2,
           Wr3, Wn3, b3, Wlin, blin, Wc, bc):
    zeros = jnp.zeros((N, DH), _f32)
    ones = jnp.ones((PC, DH), _f32)
    b1r = b1.reshape(1, D)
    b2r = b2.reshape(1, D)
    b3r = b3.reshape(1, D)

    def gnn(x, ei, batch, root):
        src, dst = ei[0], ei[1]
        x0, x1 = _split(x)
        a0, a1 = _scatter(x0, x1, src, dst, zeros)
        h, h0, h1 = _layer(a0, a1, x, Wn1, Wr1, b1r)
        a0, a1 = _scatter(h0, h1, src, dst, zeros)
        h, h0, h1 = _layer(a0, a1, h, Wn2, Wr2, b2r)
        a0, a1 = _scatter(h0, h1, src, dst, zeros)
        h3 = _layer3(a0, a1, h, Wn3, Wr3, b3r)
        return _pool(h3, batch, root, ones, zeros)

    p0s, p1s, cs, rs = gnn(x_s, edge_index_s, batch_s, root_n_id_s)
    p0t, p1t, ct, rt = gnn(x_t, edge_index_t, batch_t, root_n_id_t)

    logits, loss11 = _head(p0s, p1s, cs, rs, p0t, p1t, ct, rt,
                           Wlin, blin.reshape(1, D), Wc, bc.reshape(1, 2),
                           labels.reshape(B, 1))
    return (loss11[0, 0], logits)
